# baseline (speedup 1.0000x reference)
import functools

import jax
import jax.numpy as jnp
from jax.experimental import pallas as pl
import jax.experimental.pallas.tpu as pltpu


def _gat_body(x_ref, xt_ref, adj_ref, w_ref, a_ref, out_ref, *, row_tile):
    i = pl.program_id(0)
    d = w_ref.shape[1]

    neg_log2e = jnp.float32(-1.4426950408889634)

    w_h = jnp.dot(x_ref[:], w_ref[:], preferred_element_type=jnp.float32)
    g = jax.lax.dot_general(
        a_ref[:, d:] * neg_log2e, w_h, (((1,), (1,)), ((), ())),
        preferred_element_type=jnp.float32)

    wh_tile = jnp.dot(xt_ref[:], w_ref[:], preferred_element_type=jnp.float32)
    f = jnp.sum(wh_tile * (a_ref[0, :d] * neg_log2e)[None, :],
                axis=1, keepdims=True)
    t = f + g
    u = jnp.minimum(t, 0.2 * t)
    e = jnp.where(adj_ref[:] != 0, jnp.exp2(u), 0.0)
    num = jnp.dot(e, w_h, preferred_element_type=jnp.float32)
    denom = jnp.sum(e, axis=1, keepdims=True)
    r = num / denom
    out_ref[:] = jnp.where(r > 0, r, jnp.exp(jnp.minimum(r, 0.0)) - 1.0)


def kernel(input, adj_mat, weights, a_values):
    n, in_dim = input.shape
    out_dim = weights.shape[1]
    row_tile = 512
    grid = (n // row_tile,)

    return pl.pallas_call(
        functools.partial(_gat_body, row_tile=row_tile),
        grid=grid,
        in_specs=[
            pl.BlockSpec((n, in_dim), lambda i: (0, 0)),
            pl.BlockSpec((row_tile, in_dim), lambda i: (i, 0)),
            pl.BlockSpec((row_tile, n), lambda i: (i, 0)),
            pl.BlockSpec((in_dim, out_dim), lambda i: (0, 0)),
            pl.BlockSpec((1, 2 * out_dim), lambda i: (0, 0)),
        ],
        out_specs=pl.BlockSpec((row_tile, out_dim), lambda i: (i, 0)),
        out_shape=jax.ShapeDtypeStruct((n, out_dim), jnp.float32),
        compiler_params=pltpu.CompilerParams(
            dimension_semantics=("parallel",)),
    )(input, input, adj_mat, weights, a_values)


# speedup vs baseline: 1.1015x; 1.1015x over previous
import functools

import jax
import jax.numpy as jnp
from jax.experimental import pallas as pl
import jax.experimental.pallas.tpu as pltpu


def _gat_body(x_ref, adj_ref, w_ref, a_ref, out_ref, wh_ref, g_ref, *, row_tile):
    i = pl.program_id(0)
    d = w_ref.shape[1]

    neg_log2e = jnp.float32(-1.4426950408889634)

    @pl.when(i == 0)
    def _():
        wh = jnp.dot(x_ref[:], w_ref[:], preferred_element_type=jnp.float32)
        wh_ref[:, :d] = wh
        wh_ref[:, d:] = jnp.ones_like(wh)
        g_ref[:] = jax.lax.dot_general(
            a_ref[:, d:] * neg_log2e, wh, (((1,), (1,)), ((), ())),
            preferred_element_type=jnp.float32)

    wh_tile = wh_ref[pl.ds(i * row_tile, row_tile), :d]
    f = jnp.sum(wh_tile * (a_ref[0, :d] * neg_log2e)[None, :],
                axis=1, keepdims=True)
    t = f + g_ref[:]
    u = jnp.minimum(t, 0.2 * t)
    e = jnp.where(adj_ref[:] != 0, jnp.exp2(u), 0.0)
    num_ext = jnp.dot(e, wh_ref[:], preferred_element_type=jnp.float32)
    num = num_ext[:, :d]
    denom = num_ext[:, d:d + 1]
    r = num / denom
    out_ref[:] = jnp.where(r > 0, r, jnp.exp(jnp.minimum(r, 0.0)) - 1.0)


def kernel(input, adj_mat, weights, a_values):
    n, in_dim = input.shape
    out_dim = weights.shape[1]
    row_tile = 512
    grid = (n // row_tile,)

    return pl.pallas_call(
        functools.partial(_gat_body, row_tile=row_tile),
        grid=grid,
        in_specs=[
            pl.BlockSpec((n, in_dim), lambda i: (0, 0)),
            pl.BlockSpec((row_tile, n), lambda i: (i, 0)),
            pl.BlockSpec((in_dim, out_dim), lambda i: (0, 0)),
            pl.BlockSpec((1, 2 * out_dim), lambda i: (0, 0)),
        ],
        out_specs=pl.BlockSpec((row_tile, out_dim), lambda i: (i, 0)),
        out_shape=jax.ShapeDtypeStruct((n, out_dim), jnp.float32),
        scratch_shapes=[
            pltpu.VMEM((n, 2 * out_dim), jnp.float32),
            pltpu.VMEM((1, n), jnp.float32),
        ],
    )(input, adj_mat, weights, a_values)
